# baseline (speedup 1.0000x reference)
import functools
import jax
import jax.numpy as jnp
from jax import lax
from jax.experimental import pallas as pl
from jax.experimental.pallas import tpu as pltpu
from jax.experimental.pallas import tpu_sc as plsc

N = 10000
D = 128
E = 320000
NC = 2
NS = 16
NW = NC * NS
CH = 128
EPT = 10240
E_PAD = EPT * NW
NCH = EPT // CH
N2 = 10240
RPT = N2 // NS
RQ = 128
NB = 4
MCH = 64
MNCH = EPT // MCH

_mesh = plsc.VectorSubcoreMesh(core_axis_name="c", subcore_axis_name="s")
_sc_params = pltpu.CompilerParams(needs_layout_passes=False)


@functools.partial(
    pl.kernel,
    out_type=jax.ShapeDtypeStruct((NW * N2,), jnp.float32),
    mesh=_mesh,
    compiler_params=_sc_params,
    scratch_types=[
        pltpu.VMEM((N2,), jnp.float32),
        pltpu.VMEM((MNCH, MCH), jnp.int32),
        pltpu.VMEM((MNCH, MCH), jnp.float32),
    ],
)
def _deg_kernel(dst_hbm, ew_hbm, parts_hbm, degv, dstv, ewv):
    cid = lax.axis_index("c")
    sid = lax.axis_index("s")
    wid = sid * NC + cid

    pltpu.sync_copy(dst_hbm.at[wid], dstv)
    pltpu.sync_copy(ew_hbm.at[wid], ewv)

    def zero(i, _):
        degv[pl.ds(i * 16, 16)] = jnp.zeros((16,), jnp.float32)
        return 0

    lax.fori_loop(0, N2 // 16, zero, 0)

    @plsc.parallel_loop(0, MNCH, 1, unroll=4)
    def chunk(j):
        for g in range(MCH // 16):
            idx = dstv[j, pl.ds(g * 16, 16)]
            w = ewv[j, pl.ds(g * 16, 16)]
            plsc.addupdate_scatter(degv, [idx], w)
    pltpu.sync_copy(degv, parts_hbm.at[pl.ds(wid * N2, N2)])


def _msg_body(write_dfull, xw_hbm, dinv_hbm, src_hbm, dst_hbm, ew_hbm,
              p_hbm, dfull_hbm, acc, *scratch):
    srcb = scratch[0:NB]
    dstb = scratch[NB:2 * NB]
    ewb = scratch[2 * NB:3 * NB]
    rowsb = scratch[3 * NB:4 * NB]
    wbuf = scratch[4 * NB]
    dinv_v = scratch[4 * NB + 1]
    semi = scratch[4 * NB + 2:4 * NB + 2 + NB]
    semg = scratch[4 * NB + 2 + NB:4 * NB + 2 + 2 * NB]
    sems = scratch[4 * NB + 2 + 2 * NB:4 * NB + 2 + 3 * NB]

    cid = lax.axis_index("c")
    sid = lax.axis_index("s")
    wid = sid * NC + cid

    pltpu.sync_copy(dinv_hbm, dinv_v)

    def zb(i, _):
        for k in range(8):
            rowsb[0][i, pl.ds(k * 16, 16)] = jnp.zeros((16,), jnp.float32)
        return 0

    lax.fori_loop(0, MCH, zb, 0)
    for q in range(RPT // MCH):
        pltpu.sync_copy(rowsb[0], acc.at[pl.ds(sid * RPT + q * MCH, MCH)])

    if write_dfull:
        for q in range(320 // MCH):
            def fill(r, _):
                ridx = jnp.full((16,), wid * 320 + q * MCH + r, jnp.int32)
                dv = plsc.load_gather(dinv_v, [ridx])
                for k in range(8):
                    rowsb[0][r, pl.ds(k * 16, 16)] = dv
                return 0

            lax.fori_loop(0, MCH, fill, 0)
            pltpu.sync_copy(rowsb[0],
                            dfull_hbm.at[pl.ds(wid * 320 + q * MCH, MCH)])

    plsc.subcore_barrier()

    def issue_idx(j, b):
        pltpu.async_copy(src_hbm.at[wid, j], srcb[b], semi[b])
        pltpu.async_copy(dst_hbm.at[wid, j], dstb[b], semi[b])
        pltpu.async_copy(ew_hbm.at[wid, j], ewb[b], semi[b])

    def wait_idx(b):
        pltpu.make_async_copy(src_hbm.at[wid, 0], srcb[b], semi[b]).wait()
        pltpu.make_async_copy(dst_hbm.at[wid, 0], dstb[b], semi[b]).wait()
        pltpu.make_async_copy(ew_hbm.at[wid, 0], ewb[b], semi[b]).wait()

    def issue_g(b):
        pltpu.async_copy(acc.at[srcb[b]], rowsb[b], semg[b])

    def wait_g(b):
        pltpu.make_async_copy(acc.at[srcb[b]], rowsb[b], semg[b]).wait()

    def process(b):
        buf = rowsb[b]
        for g in range(MCH // 16):
            idx = srcb[b][pl.ds(g * 16, 16)]
            dsv = plsc.load_gather(dinv_v, [idx])
            wbuf[pl.ds(g * 16, 16)] = ewb[b][pl.ds(g * 16, 16)] * dsv

        @plsc.parallel_loop(0, MCH, 1, unroll=8)
        def srow(e):
            wv = plsc.load_gather(wbuf, [jnp.full((16,), e, jnp.int32)])
            for k in range(8):
                buf[e, pl.ds(k * 16, 16)] = buf[e, pl.ds(k * 16, 16)] * wv

        pltpu.async_copy(buf, acc.at[dstb[b]], sems[b], add=True)

    def wait_s(b):
        pltpu.make_async_copy(rowsb[b], acc.at[dstb[b]], sems[b]).wait()

    nclamp = MNCH - 1
    for b in range(NB):
        issue_idx(b, b)
    for b in range(NB - 1):
        wait_idx(b)
        issue_g(b)

    def lap(i, first):
        for b in range(NB):
            j = i * NB + b
            wait_g(b)
            process(b)
            issue_idx(jnp.minimum(j + NB, nclamp), b)
            bn = (b + NB - 1) % NB
            wait_idx(bn)
            if not (first and b == 0):
                wait_s(bn)
            issue_g(bn)

    lap(0, True)

    def pipe(i, _):
        lap(i, False)
        return 0

    lax.fori_loop(1, MNCH // NB, pipe, 0)
    wait_idx(NB - 1)
    for b in range(NB - 1):
        wait_g(b)
    wait_s(NB - 1)

    plsc.subcore_barrier()

    for q in range(RPT // RQ):
        r0 = sid * RPT + q * RQ
        pltpu.sync_copy(acc.at[pl.ds(r0, RQ)], p_hbm.at[cid, pl.ds(r0, RQ)])


_msg_scratch = (
    [pltpu.VMEM_SHARED((N2, D), jnp.float32)]
    + [pltpu.VMEM((MCH,), jnp.int32) for _ in range(NB)]
    + [pltpu.VMEM((MCH,), jnp.int32) for _ in range(NB)]
    + [pltpu.VMEM((MCH,), jnp.float32) for _ in range(NB)]
    + [pltpu.VMEM((MCH, D), jnp.float32) for _ in range(NB)]
    + [pltpu.VMEM((MCH,), jnp.float32)]
    + [pltpu.VMEM((N2,), jnp.float32)]
    + [pltpu.SemaphoreType.DMA for _ in range(3 * NB)]
)

_msg_kernel_c1 = functools.partial(
    pl.kernel,
    out_type=(jax.ShapeDtypeStruct((NC, N2, D), jnp.float32),
              jax.ShapeDtypeStruct((N2, D), jnp.float32)),
    mesh=_mesh,
    compiler_params=_sc_params,
    scratch_types=_msg_scratch,
)(functools.partial(_msg_body, True))


def _msg_body2(xw_hbm, dinv_hbm, src_hbm, dst_hbm, ew_hbm, p_hbm, *rest):
    _msg_body(False, xw_hbm, dinv_hbm, src_hbm, dst_hbm, ew_hbm,
              p_hbm, None, *rest)


_msg_kernel_c2 = functools.partial(
    pl.kernel,
    out_type=jax.ShapeDtypeStruct((NC, N2, D), jnp.float32),
    mesh=_mesh,
    compiler_params=_sc_params,
    scratch_types=_msg_scratch,
)(_msg_body2)


def _tc_mm_body(x_ref, w_ref, out_ref):
    out_ref[...] = jnp.dot(x_ref[...], w_ref[...],
                           preferred_element_type=jnp.float32)


def _tc_mm(x, w):
    return pl.pallas_call(
        _tc_mm_body,
        out_shape=jax.ShapeDtypeStruct((N2, D), jnp.float32),
    )(x, w)


def _tc_dinv_body(parts_ref, dinv_ref):
    deg = 1.0 + jnp.sum(parts_ref[...], axis=0)
    dinv_ref[...] = lax.rsqrt(deg)


def _tc_dinv(parts3):
    return pl.pallas_call(
        _tc_dinv_body,
        out_shape=jax.ShapeDtypeStruct((N2 // 128, 128), jnp.float32),
    )(parts3)


def _tc_d_body(d_ref, p_ref, xw_ref, b_ref, a_ref, w2_ref, out_ref):
    dv = d_ref[...]
    z = dv * (p_ref[0] + p_ref[1]) + dv * dv * xw_ref[...] + b_ref[...]
    a = a_ref[0, 0]
    z = jnp.where(z >= 0, z, a * z)
    out_ref[...] = jnp.dot(z, w2_ref[...], preferred_element_type=jnp.float32)


def _tc_d(dmat, p, xw, b, a, w2):
    return pl.pallas_call(
        _tc_d_body,
        out_shape=jax.ShapeDtypeStruct((N2, D), jnp.float32),
    )(dmat, p, xw, b, a, w2)


def _tc_f_body(d_ref, p_ref, xw_ref, b_ref, a_ref, g_ref, be_ref, out_ref):
    dv = d_ref[...]
    z = dv * (p_ref[0] + p_ref[1]) + dv * dv * xw_ref[...] + b_ref[...]
    a = a_ref[0, 0]
    z = jnp.where(z >= 0, z, a * z)
    z = z[:N]
    mu = jnp.mean(z, axis=0, keepdims=True)
    var = jnp.mean((z - mu) * (z - mu), axis=0, keepdims=True)
    h = (z - mu) * lax.rsqrt(var + 1e-5) * g_ref[...] + be_ref[...]
    out_ref[:, 0:D] = h
    out_ref[:, D:2 * D] = h


def _tc_f(dmat, p, xw, b, a, gamma, beta):
    return pl.pallas_call(
        _tc_f_body,
        out_shape=jax.ShapeDtypeStruct((N, 2 * D), jnp.float32),
    )(dmat, p, xw, b, a, gamma, beta)


def kernel(x, edge_weight, W1, b1, prelu_a, W2, b2, gamma, beta, edge_index):
    src = edge_index[0]
    dst = edge_index[1]
    pad = E_PAD - E
    zi = jnp.arange(pad, dtype=jnp.int32) % N
    srcp = jnp.concatenate([src, zi]).reshape(NW, MNCH, MCH)
    dstp = jnp.concatenate([dst, zi]).reshape(NW, MNCH, MCH)
    ewp = jnp.concatenate(
        [edge_weight, jnp.zeros((pad,), jnp.float32)]).reshape(NW, MNCH, MCH)

    xp = jnp.concatenate([x, jnp.zeros((N2 - N, D), jnp.float32)])

    parts = _deg_kernel(dstp, ewp)
    parts3 = parts.reshape(NW, N2 // 128, 128)
    xw1 = _tc_mm(xp, W1)
    dinv2d = _tc_dinv(parts3)
    dinv_flat = dinv2d.reshape(N2)

    p1, dfull = _msg_kernel_c1(xw1, dinv_flat, srcp, dstp, ewp)

    a2 = prelu_a.reshape(1, 1)
    xw2 = _tc_d(dfull, p1, xw1, b1.reshape(1, D), a2, W2)

    p2 = _msg_kernel_c2(xw2, dinv_flat, srcp, dstp, ewp)
    out = _tc_f(dfull, p2, xw2, b2.reshape(1, D), a2,
                gamma.reshape(1, D), beta.reshape(1, D))
    return out


# speedup vs baseline: 1.1032x; 1.1032x over previous
import functools
import jax
import jax.numpy as jnp
from jax import lax
from jax.experimental import pallas as pl
from jax.experimental.pallas import tpu as pltpu
from jax.experimental.pallas import tpu_sc as plsc

N = 10000
D = 128
E = 320000
NC = 2
NS = 16
NW = NC * NS
CH = 128
EPT = 10240
E_PAD = EPT * NW
NCH = EPT // CH
N2 = 10240
RPT = N2 // NS
RQ = 128
NB = 4
MCH = 64
MNCH = EPT // MCH

_mesh = plsc.VectorSubcoreMesh(core_axis_name="c", subcore_axis_name="s")
_sc_params = pltpu.CompilerParams(needs_layout_passes=False)


@functools.partial(
    pl.kernel,
    out_type=jax.ShapeDtypeStruct((NW * N2,), jnp.float32),
    mesh=_mesh,
    compiler_params=_sc_params,
    scratch_types=[
        pltpu.VMEM((N2,), jnp.float32),
        pltpu.VMEM((MNCH, MCH), jnp.int32),
        pltpu.VMEM((MNCH, MCH), jnp.float32),
    ],
)
def _deg_kernel(dst_hbm, ew_hbm, parts_hbm, degv, dstv, ewv):
    cid = lax.axis_index("c")
    sid = lax.axis_index("s")
    wid = sid * NC + cid

    pltpu.sync_copy(dst_hbm.at[wid], dstv)
    pltpu.sync_copy(ew_hbm.at[wid], ewv)

    def zero(i, _):
        degv[pl.ds(i * 16, 16)] = jnp.zeros((16,), jnp.float32)
        return 0

    lax.fori_loop(0, N2 // 16, zero, 0)

    @plsc.parallel_loop(0, MNCH, 1, unroll=4)
    def chunk(j):
        for g in range(MCH // 16):
            idx = dstv[j, pl.ds(g * 16, 16)]
            w = ewv[j, pl.ds(g * 16, 16)]
            plsc.addupdate_scatter(degv, [idx], w)
    pltpu.sync_copy(degv, parts_hbm.at[pl.ds(wid * N2, N2)])


def _msg_body(write_dfull, xw_hbm, dinv_hbm, src_hbm, dst_hbm, ew_hbm,
              p_hbm, dfull_hbm, acc, *scratch):
    srcb = scratch[0:NB]
    dstb = scratch[NB:2 * NB]
    ewb = scratch[2 * NB:3 * NB]
    rowsb = scratch[3 * NB:4 * NB]
    wbuf = scratch[4 * NB]
    dinv_v = scratch[4 * NB + 1]
    semi = scratch[4 * NB + 2:4 * NB + 2 + NB]
    semg = scratch[4 * NB + 2 + NB:4 * NB + 2 + 2 * NB]
    sems = scratch[4 * NB + 2 + 2 * NB:4 * NB + 2 + 3 * NB]

    cid = lax.axis_index("c")
    sid = lax.axis_index("s")
    wid = sid * NC + cid

    pltpu.sync_copy(dinv_hbm, dinv_v)

    def zb(i, _):
        for k in range(8):
            rowsb[0][i, pl.ds(k * 16, 16)] = jnp.zeros((16,), jnp.float32)
        return 0

    lax.fori_loop(0, MCH, zb, 0)
    for q in range(RPT // MCH):
        pltpu.async_copy(rowsb[0], acc.at[pl.ds(sid * RPT + q * MCH, MCH)],
                         semg[0])
    for q in range(RPT // MCH):
        pltpu.make_async_copy(
            rowsb[0], acc.at[pl.ds(sid * RPT, MCH)], semg[0]).wait()

    if write_dfull:
        nq = 320 // MCH
        for q in range(nq):
            buf = rowsb[q % 2]

            def fill(r, _):
                ridx = jnp.full((16,), wid * 320 + q * MCH + r, jnp.int32)
                dv = plsc.load_gather(dinv_v, [ridx])
                for k in range(8):
                    buf[r, pl.ds(k * 16, 16)] = dv
                return 0

            if q >= 2:
                pltpu.make_async_copy(
                    buf, dfull_hbm.at[pl.ds(0, MCH)], semg[1]).wait()
            lax.fori_loop(0, MCH, fill, 0)
            pltpu.async_copy(
                buf, dfull_hbm.at[pl.ds(wid * 320 + q * MCH, MCH)], semg[1])
        for q in range(min(nq, 2)):
            pltpu.make_async_copy(
                rowsb[q], dfull_hbm.at[pl.ds(0, MCH)], semg[1]).wait()

    plsc.subcore_barrier()

    def issue_idx(j, b):
        pltpu.async_copy(src_hbm.at[wid, j], srcb[b], semi[b])
        pltpu.async_copy(dst_hbm.at[wid, j], dstb[b], semi[b])
        pltpu.async_copy(ew_hbm.at[wid, j], ewb[b], semi[b])

    def wait_idx(b):
        pltpu.make_async_copy(src_hbm.at[wid, 0], srcb[b], semi[b]).wait()
        pltpu.make_async_copy(dst_hbm.at[wid, 0], dstb[b], semi[b]).wait()
        pltpu.make_async_copy(ew_hbm.at[wid, 0], ewb[b], semi[b]).wait()

    def issue_g(b):
        pltpu.async_copy(xw_hbm.at[srcb[b]], rowsb[b], semg[b])

    def wait_g(b):
        pltpu.make_async_copy(xw_hbm.at[srcb[b]], rowsb[b], semg[b]).wait()

    def process(b):
        buf = rowsb[b]
        for g in range(MCH // 16):
            idx = srcb[b][pl.ds(g * 16, 16)]
            dsv = plsc.load_gather(dinv_v, [idx])
            wbuf[pl.ds(g * 16, 16)] = ewb[b][pl.ds(g * 16, 16)] * dsv

        @plsc.parallel_loop(0, MCH, 1, unroll=8)
        def srow(e):
            wv = plsc.load_gather(wbuf, [jnp.full((16,), e, jnp.int32)])
            for k in range(8):
                buf[e, pl.ds(k * 16, 16)] = buf[e, pl.ds(k * 16, 16)] * wv

        pltpu.async_copy(buf, acc.at[dstb[b]], sems[b], add=True)

    def wait_s(b):
        pltpu.make_async_copy(rowsb[b], acc.at[dstb[b]], sems[b]).wait()

    nclamp = MNCH - 1
    for b in range(NB):
        issue_idx(b, b)
    for b in range(NB - 1):
        wait_idx(b)
        issue_g(b)

    def lap(i, first):
        for b in range(NB):
            j = i * NB + b
            wait_g(b)
            process(b)
            issue_idx(jnp.minimum(j + NB, nclamp), b)
            bn = (b + NB - 1) % NB
            wait_idx(bn)
            if not (first and b == 0):
                wait_s(bn)
            issue_g(bn)

    lap(0, True)

    def pipe(i, _):
        lap(i, False)
        return 0

    lax.fori_loop(1, MNCH // NB, pipe, 0)
    wait_idx(NB - 1)
    for b in range(NB - 1):
        wait_g(b)
    wait_s(NB - 1)

    plsc.subcore_barrier()

    for q in range(RPT // RQ):
        r0 = sid * RPT + q * RQ
        pltpu.sync_copy(acc.at[pl.ds(r0, RQ)], p_hbm.at[cid, pl.ds(r0, RQ)])


_msg_scratch = (
    [pltpu.VMEM_SHARED((N2, D), jnp.float32)]
    + [pltpu.VMEM((MCH,), jnp.int32) for _ in range(NB)]
    + [pltpu.VMEM((MCH,), jnp.int32) for _ in range(NB)]
    + [pltpu.VMEM((MCH,), jnp.float32) for _ in range(NB)]
    + [pltpu.VMEM((MCH, D), jnp.float32) for _ in range(NB)]
    + [pltpu.VMEM((MCH,), jnp.float32)]
    + [pltpu.VMEM((N2,), jnp.float32)]
    + [pltpu.SemaphoreType.DMA for _ in range(3 * NB)]
)

_msg_kernel_c1 = functools.partial(
    pl.kernel,
    out_type=(jax.ShapeDtypeStruct((NC, N2, D), jnp.float32),
              jax.ShapeDtypeStruct((N2, D), jnp.float32)),
    mesh=_mesh,
    compiler_params=_sc_params,
    scratch_types=_msg_scratch,
)(functools.partial(_msg_body, True))


def _msg_body2(xw_hbm, dinv_hbm, src_hbm, dst_hbm, ew_hbm, p_hbm, *rest):
    _msg_body(False, xw_hbm, dinv_hbm, src_hbm, dst_hbm, ew_hbm,
              p_hbm, None, *rest)


_msg_kernel_c2 = functools.partial(
    pl.kernel,
    out_type=jax.ShapeDtypeStruct((NC, N2, D), jnp.float32),
    mesh=_mesh,
    compiler_params=_sc_params,
    scratch_types=_msg_scratch,
)(_msg_body2)


def _tc_mm_body(x_ref, w_ref, out_ref):
    out_ref[...] = jnp.dot(x_ref[...], w_ref[...],
                           preferred_element_type=jnp.float32)


def _tc_mm(x, w):
    return pl.pallas_call(
        _tc_mm_body,
        out_shape=jax.ShapeDtypeStruct((N2, D), jnp.float32),
    )(x, w)


def _tc_dinv_body(parts_ref, dinv_ref):
    deg = 1.0 + jnp.sum(parts_ref[...], axis=0)
    dinv_ref[...] = lax.rsqrt(deg)


def _tc_dinv(parts3):
    return pl.pallas_call(
        _tc_dinv_body,
        out_shape=jax.ShapeDtypeStruct((N2 // 128, 128), jnp.float32),
    )(parts3)


def _tc_d_body(d_ref, p_ref, xw_ref, b_ref, a_ref, w2_ref, out_ref):
    dv = d_ref[...]
    z = dv * (p_ref[0] + p_ref[1]) + dv * dv * xw_ref[...] + b_ref[...]
    a = a_ref[0, 0]
    z = jnp.where(z >= 0, z, a * z)
    out_ref[...] = jnp.dot(z, w2_ref[...], preferred_element_type=jnp.float32)


def _tc_d(dmat, p, xw, b, a, w2):
    return pl.pallas_call(
        _tc_d_body,
        out_shape=jax.ShapeDtypeStruct((N2, D), jnp.float32),
    )(dmat, p, xw, b, a, w2)


def _tc_f_body(d_ref, p_ref, xw_ref, b_ref, a_ref, g_ref, be_ref, out_ref):
    dv = d_ref[...]
    z = dv * (p_ref[0] + p_ref[1]) + dv * dv * xw_ref[...] + b_ref[...]
    a = a_ref[0, 0]
    z = jnp.where(z >= 0, z, a * z)
    z = z[:N]
    mu = jnp.mean(z, axis=0, keepdims=True)
    var = jnp.mean((z - mu) * (z - mu), axis=0, keepdims=True)
    h = (z - mu) * lax.rsqrt(var + 1e-5) * g_ref[...] + be_ref[...]
    out_ref[:, 0:D] = h
    out_ref[:, D:2 * D] = h


def _tc_f(dmat, p, xw, b, a, gamma, beta):
    return pl.pallas_call(
        _tc_f_body,
        out_shape=jax.ShapeDtypeStruct((N, 2 * D), jnp.float32),
    )(dmat, p, xw, b, a, gamma, beta)


def kernel(x, edge_weight, W1, b1, prelu_a, W2, b2, gamma, beta, edge_index):
    src = edge_index[0]
    dst = edge_index[1]
    pad = E_PAD - E
    zi = jnp.arange(pad, dtype=jnp.int32) % N
    srcp = jnp.concatenate([src, zi]).reshape(NW, MNCH, MCH)
    dstp = jnp.concatenate([dst, zi]).reshape(NW, MNCH, MCH)
    ewp = jnp.concatenate(
        [edge_weight, jnp.zeros((pad,), jnp.float32)]).reshape(NW, MNCH, MCH)

    xp = jnp.concatenate([x, jnp.zeros((N2 - N, D), jnp.float32)])

    parts = _deg_kernel(dstp, ewp)
    parts3 = parts.reshape(NW, N2 // 128, 128)
    xw1 = _tc_mm(xp, W1)
    dinv2d = _tc_dinv(parts3)
    dinv_flat = dinv2d.reshape(N2)

    p1, dfull = _msg_kernel_c1(xw1, dinv_flat, srcp, dstp, ewp)

    a2 = prelu_a.reshape(1, 1)
    xw2 = _tc_d(dfull, p1, xw1, b1.reshape(1, D), a2, W2)

    p2 = _msg_kernel_c2(xw2, dinv_flat, srcp, dstp, ewp)
    out = _tc_f(dfull, p2, xw2, b2.reshape(1, D), a2,
                gamma.reshape(1, D), beta.reshape(1, D))
    return out
